# baseline (speedup 1.0000x reference)
import functools

import jax
import jax.numpy as jnp
from jax import lax
from jax.experimental import pallas as pl
from jax.experimental.pallas import tpu as pltpu
from jax.experimental.pallas import tpu_sc as plsc

_NUM_NEI = (16, 32)
_HD = 64
_SCALE = _HD ** (-0.5)

_NW = 32
_CH = 64


def _gelu(x):
    return 0.5 * x * (1.0 + lax.erf(x * (2.0 ** -0.5)))


def _mm(a, b):
    return lax.dot_general(a, b, (((1,), (0,)), ((), ())),
                           preferred_element_type=jnp.float32)


def _topk_body(pos_ref, pos_t_ref, idx_ref, *, blkq, n, k):
    b = pl.program_id(0)
    qp = pos_ref[0]
    pt = pos_t_ref[0]
    kn = (pt * pt).sum(axis=0, keepdims=True)
    scores = jnp.broadcast_to(kn, (blkq, n))
    for c in range(3):
        scores = scores - 2.0 * (qp[:, c:c + 1] * pt[c:c + 1, :])
    iota = lax.broadcasted_iota(jnp.int32, (blkq, n), 1).astype(jnp.float32)
    cols = []
    big = jnp.float32(jnp.inf)
    fn = jnp.float32(n)
    for _ in range(k):
        m = jnp.min(scores, axis=1, keepdims=True)
        sel = jnp.where(scores == m, iota, fn)
        idx = jnp.min(sel, axis=1, keepdims=True)
        cols.append(idx)
        scores = jnp.where(iota == idx, big, scores)
    mat = jnp.concatenate(cols, axis=1).astype(jnp.int32)
    idx_ref[0] = mat + b * n


def _topk_call(pos, pos_t):
    B, N, _ = pos.shape
    blkq = 128
    nb = N // blkq
    k = _NUM_NEI[1]
    return pl.pallas_call(
        functools.partial(_topk_body, blkq=blkq, n=N, k=k),
        grid=(B, nb),
        in_specs=[
            pl.BlockSpec((1, blkq, 3), lambda b, i: (b, i, 0)),
            pl.BlockSpec((1, 3, N), lambda b, i: (b, 0, 0)),
        ],
        out_specs=pl.BlockSpec((1, blkq, k), lambda b, i: (b, i, 0)),
        out_shape=jax.ShapeDtypeStruct((B, N, k), jnp.int32),
    )(pos, pos_t)


def _qkv_body(x_ref, w_ref, pp_ref, q_ref, t16_ref, t32_ref):
    y = _mm(x_ref[...], w_ref[...])
    yb = y.astype(jnp.bfloat16)
    q_ref[...] = y[:, 0:256]
    t16_ref[:, 0:128] = yb[:, 256:384]
    t16_ref[:, 128:256] = yb[:, 512:640]
    t32_ref[:, 0:128] = yb[:, 384:512]
    t32_ref[:, 128:256] = yb[:, 640:768]
    t32_ref[:, 256:384] = pp_ref[...].astype(jnp.bfloat16)
    t32_ref[:, 384:512] = jnp.zeros((pp_ref.shape[0], 128), jnp.bfloat16)


def _qkv_call(xf, w_qkv, pospad):
    M, D = xf.shape
    blk = 512
    nb = M // blk
    outs = [
        jax.ShapeDtypeStruct((M, 256), jnp.float32),
        jax.ShapeDtypeStruct((M, 256), jnp.bfloat16),
        jax.ShapeDtypeStruct((M, 512), jnp.bfloat16),
    ]
    return pl.pallas_call(
        _qkv_body,
        grid=(nb,),
        in_specs=[
            pl.BlockSpec((blk, D), lambda i: (i, 0)),
            pl.BlockSpec((D, 768), lambda i: (0, 0)),
            pl.BlockSpec((blk, 128), lambda i: (i, 0)),
        ],
        out_specs=[
            pl.BlockSpec((blk, 256), lambda i: (i, 0)),
            pl.BlockSpec((blk, 256), lambda i: (i, 0)),
            pl.BlockSpec((blk, 512), lambda i: (i, 0)),
        ],
        out_shape=outs,
    )(xf, w_qkv, pospad)


def _pipe_gather(tbl, idxv, out_ref, bufa, bufb, sema, semb, nchunk, row_base):
    def fire(c, buf, sem):
        pltpu.async_copy(tbl.at[idxv.at[c]], buf, sem)

    def drain(buf, sem):
        pltpu.make_async_copy(tbl.at[idxv.at[0]], buf, sem).wait()

    def scat(c, buf):
        pltpu.sync_copy(buf, out_ref.at[pl.ds(row_base + c * _CH, _CH)])

    fire(0, bufa, sema)

    def body(p, carry):
        c0 = 2 * p
        fire(c0 + 1, bufb, semb)
        drain(bufa, sema)
        scat(c0, bufa)
        fire(c0 + 2, bufa, sema)
        drain(bufb, semb)
        scat(c0 + 1, bufb)
        return carry

    lax.fori_loop(0, nchunk // 2 - 1, body, 0)
    c0 = nchunk - 2
    fire(c0 + 1, bufb, semb)
    drain(bufa, sema)
    scat(c0, bufa)
    drain(bufb, semb)
    scat(c0 + 1, bufb)


def _sc_gather_body(t16, t32, idx16r, idx32r, g0out, g1out,
                    idxv16, idxv32, bufa16, bufb16, bufa32, bufb32,
                    sema, semb, semc, semd, *, c16, c32):
    wid = lax.axis_index("s") * 2 + lax.axis_index("c")
    pltpu.sync_copy(idx16r.at[pl.ds(wid * c16, c16)], idxv16)
    pltpu.sync_copy(idx32r.at[pl.ds(wid * c32, c32)], idxv32)
    _pipe_gather(t16, idxv16, g0out, bufa16, bufb16, sema, semb,
                 c16, wid * c16 * _CH)
    _pipe_gather(t32, idxv32, g1out, bufa32, bufb32, semc, semd,
                 c32, wid * c32 * _CH)


def _sc_gather_call(t16, t32, idx16, idx32):
    n16 = idx16.shape[0]
    n32 = idx32.shape[0]
    c16 = n16 // (_NW * _CH)
    c32 = n32 // (_NW * _CH)
    idx16r = idx16.reshape(-1, _CH)
    idx32r = idx32.reshape(-1, _CH)
    mesh = plsc.VectorSubcoreMesh(core_axis_name="c", subcore_axis_name="s",
                                  num_cores=2, num_subcores=16)
    fn = pl.kernel(
        functools.partial(_sc_gather_body, c16=c16, c32=c32),
        out_type=(
            jax.ShapeDtypeStruct((n16, 128), jnp.int32),
            jax.ShapeDtypeStruct((n32, 256), jnp.int32),
        ),
        mesh=mesh,
        scratch_types=[
            pltpu.VMEM((c16, _CH), jnp.int32),
            pltpu.VMEM((c32, _CH), jnp.int32),
            pltpu.VMEM((_CH, 128), jnp.int32),
            pltpu.VMEM((_CH, 128), jnp.int32),
            pltpu.VMEM((_CH, 256), jnp.int32),
            pltpu.VMEM((_CH, 256), jnp.int32),
            pltpu.SemaphoreType.DMA,
            pltpu.SemaphoreType.DMA,
            pltpu.SemaphoreType.DMA,
            pltpu.SemaphoreType.DMA,
        ],
    )
    return fn(t16, t32, idx16r, idx32r)


def _attn_body(q_ref, g0_ref, g1_ref, pos_ref,
               wp0_ref, bp0_ref, wp1_ref, bp1_ref, wproj_ref, bproj_ref,
               xc_ref, feats_ref, fsum_ref, *, blk):
    i = pl.program_id(1)
    q = q_ref[...]
    pq = pos_ref[...]
    g1 = g1_ref[...].astype(jnp.float32)
    g0 = g0_ref[...].astype(jnp.float32)
    xyzr = g1[:, 256:384].reshape(blk, 32, 128)

    lane = lax.broadcasted_iota(jnp.int32, (128, 8), 0)
    colj = lax.broadcasted_iota(jnp.int32, (128, 8), 1)
    seg = jnp.where((lane < 64) == (colj == 0), 1.0, 0.0) * jnp.where(colj < 2, 1.0, 0.0)
    rowi = lax.broadcasted_iota(jnp.int32, (8, 128), 0)
    lanej = lax.broadcasted_iota(jnp.int32, (8, 128), 1)
    segt = jnp.where((lanej < 64) == (rowi == 0), 1.0, 0.0) * jnp.where(rowi < 2, 1.0, 0.0)

    relf = (pq[:, None, :] - xyzr).reshape(blk * 32, 128)

    vrp1 = _gelu(_mm(relf, wp1_ref[...]) + bp1_ref[...])
    rel0 = relf.reshape(blk, 32, 128)[:, :16, :].reshape(blk * 16, 128)
    vrp0 = _gelu(_mm(rel0, wp0_ref[...]) + bp0_ref[...])

    outs = []
    for g, (kk2, vv2, vrp, nk) in enumerate((
            (g0[:, 0:128], g0[:, 128:256], vrp0, 16),
            (g1[:, 0:128], g1[:, 128:256], vrp1, 32))):
        qg = q[:, g * 128:(g + 1) * 128]
        prod = (qg[:, None, :] * kk2.reshape(blk, nk, 128)).reshape(blk * nk, 128)
        logits8 = _mm(prod * _SCALE + vrp, seg).reshape(blk, nk, 8)
        m = jnp.max(logits8, axis=1, keepdims=True)
        e = jnp.exp(logits8 - m)
        w3 = e * (1.0 / e.sum(axis=1, keepdims=True))
        wb = _mm(w3.reshape(blk * nk, 8), segt)
        contrib = (wb * (vv2 + vrp)).reshape(blk, nk, 128)
        outs.append(contrib.sum(axis=1))

    xc = jnp.concatenate(outs, axis=1)
    feats = _gelu(_mm(xc, wproj_ref[...]) + bproj_ref[...])
    xc_ref[...] = xc
    feats_ref[...] = feats
    part = jnp.broadcast_to(feats.sum(axis=0, keepdims=True)[None], (1, 8, 256))

    @pl.when(i == 0)
    def _():
        fsum_ref[...] = part

    @pl.when(i != 0)
    def _():
        fsum_ref[...] = fsum_ref[...] + part


def _attn_call(q, g0, g1, pospad, w_pos0, b_pos0, w_pos1, b_pos1,
               wproj, bproj, B, N):
    blk = 128
    nb = N // blk
    outs = [
        jax.ShapeDtypeStruct((B * N, 256), jnp.float32),
        jax.ShapeDtypeStruct((B * N, 256), jnp.float32),
        jax.ShapeDtypeStruct((B, 8, 256), jnp.float32),
    ]
    return pl.pallas_call(
        functools.partial(_attn_body, blk=blk),
        grid=(B, nb),
        in_specs=[
            pl.BlockSpec((blk, 256), lambda b, i: (b * (N // 128) + i, 0)),
            pl.BlockSpec((blk * 16, 256), lambda b, i: (b * (N // 128) + i, 0)),
            pl.BlockSpec((blk * 32, 512), lambda b, i: (b * (N // 128) + i, 0)),
            pl.BlockSpec((blk, 128), lambda b, i: (b * (N // 128) + i, 0)),
            pl.BlockSpec((128, 128), lambda b, i: (0, 0)),
            pl.BlockSpec((1, 128), lambda b, i: (0, 0)),
            pl.BlockSpec((128, 128), lambda b, i: (0, 0)),
            pl.BlockSpec((1, 128), lambda b, i: (0, 0)),
            pl.BlockSpec((256, 256), lambda b, i: (0, 0)),
            pl.BlockSpec((1, 256), lambda b, i: (0, 0)),
        ],
        out_specs=[
            pl.BlockSpec((blk, 256), lambda b, i: (b * (N // 128) + i, 0)),
            pl.BlockSpec((blk, 256), lambda b, i: (b * (N // 128) + i, 0)),
            pl.BlockSpec((1, 8, 256), lambda b, i: (b, 0, 0)),
        ],
        out_shape=outs,
    )(q, g0, g1, pospad, w_pos0, b_pos0, w_pos1, b_pos1, wproj, bproj)


def _msf_body(xc_ref, feats_ref, fsum_ref, w1_ref, b1_ref, w2_ref, b2_ref,
              wh_ref, bh_ref, out_ref, *, n):
    s = fsum_ref[0, 0:1, :] * (1.0 / n)
    z = _gelu(_mm(s, w1_ref[...]) + b1_ref[...])
    al = _mm(z, w2_ref[...]) + b2_ref[...]
    a0 = al[:, :128]
    a1 = al[:, 128:]
    m = jnp.maximum(a0, a1)
    e0 = jnp.exp(a0 - m)
    e1 = jnp.exp(a1 - m)
    tot = e0 + e1
    av = jnp.concatenate([e0 / tot, e1 / tot], axis=1)
    scaled = xc_ref[...] * av
    out_ref[...] = feats_ref[...] + _mm(scaled, wh_ref[...]) + bh_ref[...]


def _msf_call(xc, feats, fsum, w1, b1, w2, b2, wh, bh, B, N):
    blk = 512
    nb = N // blk
    return pl.pallas_call(
        functools.partial(_msf_body, n=N),
        grid=(B, nb),
        in_specs=[
            pl.BlockSpec((blk, 256), lambda b, i: (b * (N // 512) + i, 0)),
            pl.BlockSpec((blk, 256), lambda b, i: (b * (N // 512) + i, 0)),
            pl.BlockSpec((1, 8, 256), lambda b, i: (b, 0, 0)),
            pl.BlockSpec((256, 128), lambda b, i: (0, 0)),
            pl.BlockSpec((1, 128), lambda b, i: (0, 0)),
            pl.BlockSpec((128, 256), lambda b, i: (0, 0)),
            pl.BlockSpec((1, 256), lambda b, i: (0, 0)),
            pl.BlockSpec((256, 256), lambda b, i: (0, 0)),
            pl.BlockSpec((1, 256), lambda b, i: (0, 0)),
        ],
        out_specs=pl.BlockSpec((blk, 256), lambda b, i: (b * (N // 512) + i, 0)),
        out_shape=jax.ShapeDtypeStruct((B * N, 256), jnp.float32),
    )(xc, feats, fsum, w1, b1, w2, b2, wh, bh)


def kernel(x, pos, W_qkv, W_pos0, b_pos0, W_pos1, b_pos1,
           msf_proj_w, msf_proj_b, msf_fc1_w, msf_fc1_b,
           msf_fc2_w, msf_fc2_b, msf_head_w, msf_head_b):
    B, N, DIM = x.shape
    xf = x.reshape(B * N, DIM)
    pos_t = pos.transpose(0, 2, 1)
    pospad = jnp.pad(pos.reshape(B * N, 3), ((0, 0), (0, 125)))

    idx32_mat = _topk_call(pos, pos_t)
    idx16 = idx32_mat[:, :, :16].reshape(-1)
    idx32 = idx32_mat.reshape(-1)

    q, t16, t32 = _qkv_call(xf, W_qkv, pospad)
    M = B * N
    t16i = lax.bitcast_convert_type(t16.reshape(M, 128, 2), jnp.int32)
    t32i = lax.bitcast_convert_type(t32.reshape(M, 256, 2), jnp.int32)
    g0i, g1i = _sc_gather_call(t16i, t32i, idx16, idx32)
    g0 = lax.bitcast_convert_type(g0i, jnp.bfloat16).reshape(M * 16, 256)
    g1 = lax.bitcast_convert_type(g1i, jnp.bfloat16).reshape(M * 32, 512)

    wp0 = jnp.pad(W_pos0, ((0, 125), (0, 0)))
    wp1 = jnp.pad(W_pos1, ((0, 125), (0, 0)))
    xc, feats, fsum = _attn_call(
        q, g0, g1, pospad,
        wp0, b_pos0.reshape(1, -1), wp1, b_pos1.reshape(1, -1),
        msf_proj_w, msf_proj_b.reshape(1, -1), B, N)

    out = _msf_call(xc, feats, fsum,
                    msf_fc1_w, msf_fc1_b.reshape(1, -1),
                    msf_fc2_w, msf_fc2_b.reshape(1, -1),
                    msf_head_w, msf_head_b.reshape(1, -1), B, N)
    return out.reshape(B, N, DIM)


# speedup vs baseline: 3.6449x; 3.6449x over previous
import functools

import jax
import jax.numpy as jnp
from jax import lax
from jax.experimental import pallas as pl
from jax.experimental.pallas import tpu as pltpu
from jax.experimental.pallas import tpu_sc as plsc

_NUM_NEI = (16, 32)
_HD = 64
_SCALE = _HD ** (-0.5)

_NW = 32
_CH = 64


def _gelu(x):
    return 0.5 * x * (1.0 + lax.erf(x * (2.0 ** -0.5)))


def _mm(a, b):
    return lax.dot_general(a, b, (((1,), (0,)), ((), ())),
                           preferred_element_type=jnp.float32)


def _topk_body(pos_ref, pos_t_ref, idx_ref, *, blkq, n, k):
    b = pl.program_id(0)
    qp = pos_ref[0]
    pt = pos_t_ref[0]
    kn = (pt * pt).sum(axis=0, keepdims=True)
    scores = jnp.broadcast_to(kn, (blkq, n))
    for c in range(3):
        scores = scores - 2.0 * (qp[:, c:c + 1] * pt[c:c + 1, :])
    iota = lax.broadcasted_iota(jnp.int32, (blkq, n), 1).astype(jnp.float32)
    cols = []
    big = jnp.float32(jnp.inf)
    fn = jnp.float32(n)
    for _ in range(k):
        m = jnp.min(scores, axis=1, keepdims=True)
        sel = jnp.where(scores == m, iota, fn)
        idx = jnp.min(sel, axis=1, keepdims=True)
        cols.append(idx)
        scores = jnp.where(iota == idx, big, scores)
    mat = jnp.concatenate(cols, axis=1).astype(jnp.int32)
    idx_ref[0] = mat + b * n


def _topk_call(pos, pos_t):
    B, N, _ = pos.shape
    blkq = 128
    nb = N // blkq
    k = _NUM_NEI[1]
    return pl.pallas_call(
        functools.partial(_topk_body, blkq=blkq, n=N, k=k),
        grid=(B, nb),
        in_specs=[
            pl.BlockSpec((1, blkq, 3), lambda b, i: (b, i, 0)),
            pl.BlockSpec((1, 3, N), lambda b, i: (b, 0, 0)),
        ],
        out_specs=pl.BlockSpec((1, blkq, k), lambda b, i: (b, i, 0)),
        out_shape=jax.ShapeDtypeStruct((B, N, k), jnp.int32),
    )(pos, pos_t)


def _pack_kv(kf, vf):
    kb = lax.bitcast_convert_type(kf.astype(jnp.bfloat16).astype(jnp.float32),
                                  jnp.int32)
    vb = lax.bitcast_convert_type(vf.astype(jnp.bfloat16).astype(jnp.float32),
                                  jnp.int32)
    return jnp.bitwise_or(lax.shift_right_logical(kb, 16), vb)


def _qkv_body(x_ref, w_ref, pp_ref, q_ref, t16_ref, t32_ref):
    y = _mm(x_ref[...], w_ref[...])
    q_ref[...] = y[:, 0:256]
    t16_ref[...] = _pack_kv(y[:, 256:384], y[:, 512:640])
    t32_ref[:, 0:128] = _pack_kv(y[:, 384:512], y[:, 640:768])
    t32_ref[:, 128:256] = lax.bitcast_convert_type(pp_ref[...], jnp.int32)


def _qkv_call(xf, w_qkv, pospad):
    M, D = xf.shape
    blk = 512
    nb = M // blk
    outs = [
        jax.ShapeDtypeStruct((M, 256), jnp.float32),
        jax.ShapeDtypeStruct((M, 128), jnp.int32),
        jax.ShapeDtypeStruct((M, 256), jnp.int32),
    ]
    return pl.pallas_call(
        _qkv_body,
        grid=(nb,),
        in_specs=[
            pl.BlockSpec((blk, D), lambda i: (i, 0)),
            pl.BlockSpec((D, 768), lambda i: (0, 0)),
            pl.BlockSpec((blk, 128), lambda i: (i, 0)),
        ],
        out_specs=[
            pl.BlockSpec((blk, 256), lambda i: (i, 0)),
            pl.BlockSpec((blk, 128), lambda i: (i, 0)),
            pl.BlockSpec((blk, 256), lambda i: (i, 0)),
        ],
        out_shape=outs,
    )(xf, w_qkv, pospad)


def _pipe_gather(tbl, idxv, out_ref, bufa, bufb, sema, semb, nchunk, row_base):
    def fire(c, buf, sem):
        pltpu.async_copy(tbl.at[idxv.at[c]], buf, sem)

    def drain(buf, sem):
        pltpu.make_async_copy(tbl.at[idxv.at[0]], buf, sem).wait()

    def scat(c, buf):
        pltpu.sync_copy(buf, out_ref.at[pl.ds(row_base + c * _CH, _CH)])

    fire(0, bufa, sema)

    def body(p, carry):
        c0 = 2 * p
        fire(c0 + 1, bufb, semb)
        drain(bufa, sema)
        scat(c0, bufa)
        fire(c0 + 2, bufa, sema)
        drain(bufb, semb)
        scat(c0 + 1, bufb)
        return carry

    lax.fori_loop(0, nchunk // 2 - 1, body, 0)
    c0 = nchunk - 2
    fire(c0 + 1, bufb, semb)
    drain(bufa, sema)
    scat(c0, bufa)
    drain(bufb, semb)
    scat(c0 + 1, bufb)


def _sc_gather_body(t16, t32, idx16r, idx32r, g0out, g1out,
                    idxv16, idxv32, bufa16, bufb16, bufa32, bufb32,
                    sema, semb, semc, semd, *, c16, c32):
    wid = lax.axis_index("s") * 2 + lax.axis_index("c")
    pltpu.sync_copy(idx16r.at[pl.ds(wid * c16, c16)], idxv16)
    pltpu.sync_copy(idx32r.at[pl.ds(wid * c32, c32)], idxv32)
    _pipe_gather(t16, idxv16, g0out, bufa16, bufb16, sema, semb,
                 c16, wid * c16 * _CH)
    _pipe_gather(t32, idxv32, g1out, bufa32, bufb32, semc, semd,
                 c32, wid * c32 * _CH)


def _sc_gather_call(t16, t32, idx16, idx32):
    n16 = idx16.shape[0]
    n32 = idx32.shape[0]
    c16 = n16 // (_NW * _CH)
    c32 = n32 // (_NW * _CH)
    idx16r = idx16.reshape(-1, _CH)
    idx32r = idx32.reshape(-1, _CH)
    mesh = plsc.VectorSubcoreMesh(core_axis_name="c", subcore_axis_name="s",
                                  num_cores=2, num_subcores=16)
    fn = pl.kernel(
        functools.partial(_sc_gather_body, c16=c16, c32=c32),
        out_type=(
            jax.ShapeDtypeStruct((n16, 128), jnp.int32),
            jax.ShapeDtypeStruct((n32, 256), jnp.int32),
        ),
        mesh=mesh,
        scratch_types=[
            pltpu.VMEM((c16, _CH), jnp.int32),
            pltpu.VMEM((c32, _CH), jnp.int32),
            pltpu.VMEM((_CH, 128), jnp.int32),
            pltpu.VMEM((_CH, 128), jnp.int32),
            pltpu.VMEM((_CH, 256), jnp.int32),
            pltpu.VMEM((_CH, 256), jnp.int32),
            pltpu.SemaphoreType.DMA,
            pltpu.SemaphoreType.DMA,
            pltpu.SemaphoreType.DMA,
            pltpu.SemaphoreType.DMA,
        ],
    )
    return fn(t16, t32, idx16r, idx32r)


def _attn_body(q_ref, g0_ref, g1_ref, pos_ref,
               wp0_ref, bp0_ref, wp1_ref, bp1_ref, wproj_ref, bproj_ref,
               xc_ref, feats_ref, fsum_ref, *, blk):
    i = pl.program_id(1)
    q = q_ref[...]
    pq = pos_ref[...]
    g0i = g0_ref[...]
    g1i = g1_ref[...]
    kk0 = lax.bitcast_convert_type(lax.shift_left(g0i, 16), jnp.float32)
    vv0 = lax.bitcast_convert_type(
        jnp.bitwise_and(g0i, jnp.int32(-65536)), jnp.float32)
    g1kv = g1i[:, 0:128]
    kk1 = lax.bitcast_convert_type(lax.shift_left(g1kv, 16), jnp.float32)
    vv1 = lax.bitcast_convert_type(
        jnp.bitwise_and(g1kv, jnp.int32(-65536)), jnp.float32)
    xyzr = lax.bitcast_convert_type(g1i[:, 128:256],
                                    jnp.float32).reshape(blk, 32, 128)

    lane = lax.broadcasted_iota(jnp.int32, (128, 8), 0)
    colj = lax.broadcasted_iota(jnp.int32, (128, 8), 1)
    seg = jnp.where((lane < 64) == (colj == 0), 1.0, 0.0) * jnp.where(colj < 2, 1.0, 0.0)
    rowi = lax.broadcasted_iota(jnp.int32, (8, 128), 0)
    lanej = lax.broadcasted_iota(jnp.int32, (8, 128), 1)
    segt = jnp.where((lanej < 64) == (rowi == 0), 1.0, 0.0) * jnp.where(rowi < 2, 1.0, 0.0)

    relf = (pq[:, None, :] - xyzr).reshape(blk * 32, 128)

    vrp1 = _gelu(_mm(relf, wp1_ref[...]) + bp1_ref[...])
    rel0 = relf.reshape(blk, 32, 128)[:, :16, :].reshape(blk * 16, 128)
    vrp0 = _gelu(_mm(rel0, wp0_ref[...]) + bp0_ref[...])

    outs = []
    for g, (kk2, vv2, vrp, nk) in enumerate((
            (kk0, vv0, vrp0, 16),
            (kk1, vv1, vrp1, 32))):
        qg = q[:, g * 128:(g + 1) * 128]
        prod = (qg[:, None, :] * kk2.reshape(blk, nk, 128)).reshape(blk * nk, 128)
        logits8 = _mm(prod * _SCALE + vrp, seg).reshape(blk, nk, 8)
        m = jnp.max(logits8, axis=1, keepdims=True)
        e = jnp.exp(logits8 - m)
        w3 = e * (1.0 / e.sum(axis=1, keepdims=True))
        wb = _mm(w3.reshape(blk * nk, 8), segt)
        contrib = (wb * (vv2 + vrp)).reshape(blk, nk, 128)
        outs.append(contrib.sum(axis=1))

    xc = jnp.concatenate(outs, axis=1)
    feats = _gelu(_mm(xc, wproj_ref[...]) + bproj_ref[...])
    xc_ref[...] = xc
    feats_ref[...] = feats
    part = jnp.broadcast_to(feats.sum(axis=0, keepdims=True)[None], (1, 8, 256))

    @pl.when(i == 0)
    def _():
        fsum_ref[...] = part

    @pl.when(i != 0)
    def _():
        fsum_ref[...] = fsum_ref[...] + part


def _attn_call(q, g0, g1, pospad, w_pos0, b_pos0, w_pos1, b_pos1,
               wproj, bproj, B, N):
    blk = 128
    nb = N // blk
    outs = [
        jax.ShapeDtypeStruct((B * N, 256), jnp.float32),
        jax.ShapeDtypeStruct((B * N, 256), jnp.float32),
        jax.ShapeDtypeStruct((B, 8, 256), jnp.float32),
    ]
    return pl.pallas_call(
        functools.partial(_attn_body, blk=blk),
        grid=(B, nb),
        in_specs=[
            pl.BlockSpec((blk, 256), lambda b, i: (b * (N // 128) + i, 0)),
            pl.BlockSpec((blk * 16, 128), lambda b, i: (b * (N // 128) + i, 0)),
            pl.BlockSpec((blk * 32, 256), lambda b, i: (b * (N // 128) + i, 0)),
            pl.BlockSpec((blk, 128), lambda b, i: (b * (N // 128) + i, 0)),
            pl.BlockSpec((128, 128), lambda b, i: (0, 0)),
            pl.BlockSpec((1, 128), lambda b, i: (0, 0)),
            pl.BlockSpec((128, 128), lambda b, i: (0, 0)),
            pl.BlockSpec((1, 128), lambda b, i: (0, 0)),
            pl.BlockSpec((256, 256), lambda b, i: (0, 0)),
            pl.BlockSpec((1, 256), lambda b, i: (0, 0)),
        ],
        out_specs=[
            pl.BlockSpec((blk, 256), lambda b, i: (b * (N // 128) + i, 0)),
            pl.BlockSpec((blk, 256), lambda b, i: (b * (N // 128) + i, 0)),
            pl.BlockSpec((1, 8, 256), lambda b, i: (b, 0, 0)),
        ],
        out_shape=outs,
    )(q, g0, g1, pospad, w_pos0, b_pos0, w_pos1, b_pos1, wproj, bproj)


def _msf_body(xc_ref, feats_ref, fsum_ref, w1_ref, b1_ref, w2_ref, b2_ref,
              wh_ref, bh_ref, out_ref, *, n):
    s = fsum_ref[0, 0:1, :] * (1.0 / n)
    z = _gelu(_mm(s, w1_ref[...]) + b1_ref[...])
    al = _mm(z, w2_ref[...]) + b2_ref[...]
    a0 = al[:, :128]
    a1 = al[:, 128:]
    m = jnp.maximum(a0, a1)
    e0 = jnp.exp(a0 - m)
    e1 = jnp.exp(a1 - m)
    tot = e0 + e1
    av = jnp.concatenate([e0 / tot, e1 / tot], axis=1)
    scaled = xc_ref[...] * av
    out_ref[...] = feats_ref[...] + _mm(scaled, wh_ref[...]) + bh_ref[...]


def _msf_call(xc, feats, fsum, w1, b1, w2, b2, wh, bh, B, N):
    blk = 512
    nb = N // blk
    return pl.pallas_call(
        functools.partial(_msf_body, n=N),
        grid=(B, nb),
        in_specs=[
            pl.BlockSpec((blk, 256), lambda b, i: (b * (N // 512) + i, 0)),
            pl.BlockSpec((blk, 256), lambda b, i: (b * (N // 512) + i, 0)),
            pl.BlockSpec((1, 8, 256), lambda b, i: (b, 0, 0)),
            pl.BlockSpec((256, 128), lambda b, i: (0, 0)),
            pl.BlockSpec((1, 128), lambda b, i: (0, 0)),
            pl.BlockSpec((128, 256), lambda b, i: (0, 0)),
            pl.BlockSpec((1, 256), lambda b, i: (0, 0)),
            pl.BlockSpec((256, 256), lambda b, i: (0, 0)),
            pl.BlockSpec((1, 256), lambda b, i: (0, 0)),
        ],
        out_specs=pl.BlockSpec((blk, 256), lambda b, i: (b * (N // 512) + i, 0)),
        out_shape=jax.ShapeDtypeStruct((B * N, 256), jnp.float32),
    )(xc, feats, fsum, w1, b1, w2, b2, wh, bh)


def kernel(x, pos, W_qkv, W_pos0, b_pos0, W_pos1, b_pos1,
           msf_proj_w, msf_proj_b, msf_fc1_w, msf_fc1_b,
           msf_fc2_w, msf_fc2_b, msf_head_w, msf_head_b):
    B, N, DIM = x.shape
    xf = x.reshape(B * N, DIM)
    pos_t = pos.transpose(0, 2, 1)
    pospad = jnp.pad(pos.reshape(B * N, 3), ((0, 0), (0, 125)))

    idx32_mat = _topk_call(pos, pos_t)
    idx16 = idx32_mat[:, :, :16].reshape(-1)
    idx32 = idx32_mat.reshape(-1)

    q, t16, t32 = _qkv_call(xf, W_qkv, pospad)
    g0, g1 = _sc_gather_call(t16, t32, idx16, idx32)

    wp0 = jnp.pad(W_pos0, ((0, 125), (0, 0)))
    wp1 = jnp.pad(W_pos1, ((0, 125), (0, 0)))
    xc, feats, fsum = _attn_call(
        q, g0, g1, pospad,
        wp0, b_pos0.reshape(1, -1), wp1, b_pos1.reshape(1, -1),
        msf_proj_w, msf_proj_b.reshape(1, -1), B, N)

    out = _msf_call(xc, feats, fsum,
                    msf_fc1_w, msf_fc1_b.reshape(1, -1),
                    msf_fc2_w, msf_fc2_b.reshape(1, -1),
                    msf_head_w, msf_head_b.reshape(1, -1), B, N)
    return out.reshape(B, N, DIM)


# speedup vs baseline: 4.4732x; 1.2273x over previous
import functools

import jax
import jax.numpy as jnp
from jax import lax
from jax.experimental import pallas as pl
from jax.experimental.pallas import tpu as pltpu
from jax.experimental.pallas import tpu_sc as plsc

_NUM_NEI = (16, 32)
_HD = 64
_SCALE = _HD ** (-0.5)

_NW = 32
_CH = 64


def _gelu(x):
    return 0.5 * x * (1.0 + lax.erf(x * (2.0 ** -0.5)))


def _mm(a, b):
    return lax.dot_general(a, b, (((1,), (0,)), ((), ())),
                           preferred_element_type=jnp.float32)


def _topk_body(pos_ref, pos_t_ref, idx_ref, *, blkq, n, k, b):
    qp = pos_ref[0]
    pt = pos_t_ref[0]
    kn = (pt * pt).sum(axis=0, keepdims=True)
    scores = jnp.broadcast_to(kn, (blkq, n))
    for c in range(3):
        scores = scores - 2.0 * (qp[:, c:c + 1] * pt[c:c + 1, :])
    iota = lax.broadcasted_iota(jnp.int32, (blkq, n), 1).astype(jnp.float32)
    cols = []
    big = jnp.float32(jnp.inf)
    fn = jnp.float32(n)
    for _ in range(k):
        m = jnp.min(scores, axis=1, keepdims=True)
        sel = jnp.where(scores == m, iota, fn)
        idx = jnp.min(sel, axis=1, keepdims=True)
        cols.append(idx)
        scores = jnp.where(iota == idx, big, scores)
    mat = jnp.concatenate(cols, axis=1).astype(jnp.int32)
    idx_ref[0] = mat + b * n


def _topk_call(pos, pos_t, b):
    B, N, _ = pos.shape
    blkq = 128
    nb = N // blkq
    k = _NUM_NEI[1]
    return pl.pallas_call(
        functools.partial(_topk_body, blkq=blkq, n=N, k=k, b=b),
        grid=(nb,),
        in_specs=[
            pl.BlockSpec((1, blkq, 3), lambda i, b=b: (b, i, 0)),
            pl.BlockSpec((1, 3, N), lambda i, b=b: (b, 0, 0)),
        ],
        out_specs=pl.BlockSpec((1, blkq, k), lambda i: (0, i, 0)),
        out_shape=jax.ShapeDtypeStruct((1, N, k), jnp.int32),
    )(pos, pos_t)


def _pack_kv(kf, vf):
    kb = lax.bitcast_convert_type(kf.astype(jnp.bfloat16).astype(jnp.float32),
                                  jnp.int32)
    vb = lax.bitcast_convert_type(vf.astype(jnp.bfloat16).astype(jnp.float32),
                                  jnp.int32)
    return jnp.bitwise_or(lax.shift_right_logical(kb, 16), vb)


def _qkv_body(x_ref, w_ref, pp_ref, q_ref, t16_ref, t32_ref):
    y = _mm(x_ref[...], w_ref[...])
    q_ref[...] = y[:, 0:256]
    t16_ref[...] = _pack_kv(y[:, 256:384], y[:, 512:640])
    t32_ref[:, 0:128] = _pack_kv(y[:, 384:512], y[:, 640:768])
    t32_ref[:, 128:256] = lax.bitcast_convert_type(pp_ref[...], jnp.int32)


def _qkv_call(xf, w_qkv, pospad):
    M, D = xf.shape
    blk = 512
    nb = M // blk
    outs = [
        jax.ShapeDtypeStruct((M, 256), jnp.float32),
        jax.ShapeDtypeStruct((M, 128), jnp.int32),
        jax.ShapeDtypeStruct((M, 256), jnp.int32),
    ]
    return pl.pallas_call(
        _qkv_body,
        grid=(nb,),
        in_specs=[
            pl.BlockSpec((blk, D), lambda i: (i, 0)),
            pl.BlockSpec((D, 768), lambda i: (0, 0)),
            pl.BlockSpec((blk, 128), lambda i: (i, 0)),
        ],
        out_specs=[
            pl.BlockSpec((blk, 256), lambda i: (i, 0)),
            pl.BlockSpec((blk, 128), lambda i: (i, 0)),
            pl.BlockSpec((blk, 256), lambda i: (i, 0)),
        ],
        out_shape=outs,
    )(xf, w_qkv, pospad)


def _pipe_gather(tbl, idxv, out_ref, bufa, bufb, sema, semb, nchunk, row_base):
    def fire(c, buf, sem):
        pltpu.async_copy(tbl.at[idxv.at[c]], buf, sem)

    def drain(buf, sem):
        pltpu.make_async_copy(tbl.at[idxv.at[0]], buf, sem).wait()

    def scat(c, buf):
        pltpu.sync_copy(buf, out_ref.at[pl.ds(row_base + c * _CH, _CH)])

    fire(0, bufa, sema)

    def body(p, carry):
        c0 = 2 * p
        fire(c0 + 1, bufb, semb)
        drain(bufa, sema)
        scat(c0, bufa)
        fire(c0 + 2, bufa, sema)
        drain(bufb, semb)
        scat(c0 + 1, bufb)
        return carry

    lax.fori_loop(0, nchunk // 2 - 1, body, 0)
    c0 = nchunk - 2
    fire(c0 + 1, bufb, semb)
    drain(bufa, sema)
    scat(c0, bufa)
    drain(bufb, semb)
    scat(c0 + 1, bufb)


def _sc_gather_body(t16, t32, idx16r, idx32r, g0out, g1out,
                    idxv16, idxv32, bufa16, bufb16, bufa32, bufb32,
                    sema, semb, semc, semd, *, c16, c32):
    wid = lax.axis_index("s") * 2 + lax.axis_index("c")
    pltpu.sync_copy(idx16r.at[pl.ds(wid * c16, c16)], idxv16)
    pltpu.sync_copy(idx32r.at[pl.ds(wid * c32, c32)], idxv32)
    _pipe_gather(t16, idxv16, g0out, bufa16, bufb16, sema, semb,
                 c16, wid * c16 * _CH)
    _pipe_gather(t32, idxv32, g1out, bufa32, bufb32, semc, semd,
                 c32, wid * c32 * _CH)


def _sc_gather_call(t16, t32, idx16, idx32):
    n16 = idx16.shape[0]
    n32 = idx32.shape[0]
    c16 = n16 // (_NW * _CH)
    c32 = n32 // (_NW * _CH)
    idx16r = idx16.reshape(-1, _CH)
    idx32r = idx32.reshape(-1, _CH)
    mesh = plsc.VectorSubcoreMesh(core_axis_name="c", subcore_axis_name="s",
                                  num_cores=2, num_subcores=16)
    fn = pl.kernel(
        functools.partial(_sc_gather_body, c16=c16, c32=c32),
        out_type=(
            jax.ShapeDtypeStruct((n16, 128), jnp.int32),
            jax.ShapeDtypeStruct((n32, 256), jnp.int32),
        ),
        mesh=mesh,
        scratch_types=[
            pltpu.VMEM((c16, _CH), jnp.int32),
            pltpu.VMEM((c32, _CH), jnp.int32),
            pltpu.VMEM((_CH, 128), jnp.int32),
            pltpu.VMEM((_CH, 128), jnp.int32),
            pltpu.VMEM((_CH, 256), jnp.int32),
            pltpu.VMEM((_CH, 256), jnp.int32),
            pltpu.SemaphoreType.DMA,
            pltpu.SemaphoreType.DMA,
            pltpu.SemaphoreType.DMA,
            pltpu.SemaphoreType.DMA,
        ],
    )
    return fn(t16, t32, idx16r, idx32r)


def _attn_body(q_ref, g0_ref, g1_ref, pos_ref,
               wp0_ref, bp0_ref, wp1_ref, bp1_ref, wproj_ref, bproj_ref,
               xc_ref, feats_ref, fsum_ref, *, blk):
    i = pl.program_id(0)
    q = q_ref[...]
    pq = pos_ref[...]
    g0i = g0_ref[...]
    g1i = g1_ref[...]
    kk0 = lax.bitcast_convert_type(lax.shift_left(g0i, 16), jnp.float32)
    vv0 = lax.bitcast_convert_type(
        jnp.bitwise_and(g0i, jnp.int32(-65536)), jnp.float32)
    g1kv = g1i[:, 0:128]
    kk1 = lax.bitcast_convert_type(lax.shift_left(g1kv, 16), jnp.float32)
    vv1 = lax.bitcast_convert_type(
        jnp.bitwise_and(g1kv, jnp.int32(-65536)), jnp.float32)
    xyzr = lax.bitcast_convert_type(g1i[:, 128:256],
                                    jnp.float32).reshape(blk, 32, 128)

    lane = lax.broadcasted_iota(jnp.int32, (128, 8), 0)
    colj = lax.broadcasted_iota(jnp.int32, (128, 8), 1)
    seg = jnp.where((lane < 64) == (colj == 0), 1.0, 0.0) * jnp.where(colj < 2, 1.0, 0.0)
    rowi = lax.broadcasted_iota(jnp.int32, (8, 128), 0)
    lanej = lax.broadcasted_iota(jnp.int32, (8, 128), 1)
    segt = jnp.where((lanej < 64) == (rowi == 0), 1.0, 0.0) * jnp.where(rowi < 2, 1.0, 0.0)

    relf = (pq[:, None, :] - xyzr).reshape(blk * 32, 128)

    vrp1 = _gelu(_mm(relf, wp1_ref[...]) + bp1_ref[...])
    rel0 = relf.reshape(blk, 32, 128)[:, :16, :].reshape(blk * 16, 128)
    vrp0 = _gelu(_mm(rel0, wp0_ref[...]) + bp0_ref[...])

    outs = []
    for g, (kk2, vv2, vrp, nk) in enumerate((
            (kk0, vv0, vrp0, 16),
            (kk1, vv1, vrp1, 32))):
        qg = q[:, g * 128:(g + 1) * 128]
        prod = (qg[:, None, :] * kk2.reshape(blk, nk, 128)).reshape(blk * nk, 128)
        logits8 = _mm(prod * _SCALE + vrp, seg).reshape(blk, nk, 8)
        m = jnp.max(logits8, axis=1, keepdims=True)
        e = jnp.exp(logits8 - m)
        w3 = e * (1.0 / e.sum(axis=1, keepdims=True))
        wb = _mm(w3.reshape(blk * nk, 8), segt)
        contrib = (wb * (vv2 + vrp)).reshape(blk, nk, 128)
        outs.append(contrib.sum(axis=1))

    xc = jnp.concatenate(outs, axis=1)
    feats = _gelu(_mm(xc, wproj_ref[...]) + bproj_ref[...])
    xc_ref[...] = xc
    feats_ref[...] = feats
    part = jnp.broadcast_to(feats.sum(axis=0, keepdims=True)[None], (1, 8, 256))

    @pl.when(i == 0)
    def _():
        fsum_ref[...] = part

    @pl.when(i != 0)
    def _():
        fsum_ref[...] = fsum_ref[...] + part


def _attn_call(q, g0, g1, pospad, w_pos0, b_pos0, w_pos1, b_pos1,
               wproj, bproj, N, b):
    blk = 128
    nb = N // blk
    outs = [
        jax.ShapeDtypeStruct((N, 256), jnp.float32),
        jax.ShapeDtypeStruct((N, 256), jnp.float32),
        jax.ShapeDtypeStruct((1, 8, 256), jnp.float32),
    ]
    return pl.pallas_call(
        functools.partial(_attn_body, blk=blk),
        grid=(nb,),
        in_specs=[
            pl.BlockSpec((blk, 256), lambda i, b=b: (b * (N // 128) + i, 0)),
            pl.BlockSpec((blk * 16, 128), lambda i: (i, 0)),
            pl.BlockSpec((blk * 32, 256), lambda i: (i, 0)),
            pl.BlockSpec((blk, 128), lambda i, b=b: (b * (N // 128) + i, 0)),
            pl.BlockSpec((128, 128), lambda i: (0, 0)),
            pl.BlockSpec((1, 128), lambda i: (0, 0)),
            pl.BlockSpec((128, 128), lambda i: (0, 0)),
            pl.BlockSpec((1, 128), lambda i: (0, 0)),
            pl.BlockSpec((256, 256), lambda i: (0, 0)),
            pl.BlockSpec((1, 256), lambda i: (0, 0)),
        ],
        out_specs=[
            pl.BlockSpec((blk, 256), lambda i: (i, 0)),
            pl.BlockSpec((blk, 256), lambda i: (i, 0)),
            pl.BlockSpec((1, 8, 256), lambda i: (0, 0, 0)),
        ],
        out_shape=outs,
    )(q, g0, g1, pospad, w_pos0, b_pos0, w_pos1, b_pos1, wproj, bproj)


def _msf_body(xc_ref, feats_ref, fsum_ref, w1_ref, b1_ref, w2_ref, b2_ref,
              wh_ref, bh_ref, out_ref, *, n):
    s = fsum_ref[0, 0:1, :] * (1.0 / n)
    z = _gelu(_mm(s, w1_ref[...]) + b1_ref[...])
    al = _mm(z, w2_ref[...]) + b2_ref[...]
    a0 = al[:, :128]
    a1 = al[:, 128:]
    m = jnp.maximum(a0, a1)
    e0 = jnp.exp(a0 - m)
    e1 = jnp.exp(a1 - m)
    tot = e0 + e1
    av = jnp.concatenate([e0 / tot, e1 / tot], axis=1)
    scaled = xc_ref[...] * av
    out_ref[...] = feats_ref[...] + _mm(scaled, wh_ref[...]) + bh_ref[...]


def _msf_call(xc, feats, fsum, w1, b1, w2, b2, wh, bh, N):
    blk = 512
    nb = N // blk
    return pl.pallas_call(
        functools.partial(_msf_body, n=N),
        grid=(nb,),
        in_specs=[
            pl.BlockSpec((blk, 256), lambda i: (i, 0)),
            pl.BlockSpec((blk, 256), lambda i: (i, 0)),
            pl.BlockSpec((1, 8, 256), lambda i: (0, 0, 0)),
            pl.BlockSpec((256, 128), lambda i: (0, 0)),
            pl.BlockSpec((1, 128), lambda i: (0, 0)),
            pl.BlockSpec((128, 256), lambda i: (0, 0)),
            pl.BlockSpec((1, 256), lambda i: (0, 0)),
            pl.BlockSpec((256, 256), lambda i: (0, 0)),
            pl.BlockSpec((1, 256), lambda i: (0, 0)),
        ],
        out_specs=pl.BlockSpec((blk, 256), lambda i: (i, 0)),
        out_shape=jax.ShapeDtypeStruct((N, 256), jnp.float32),
    )(xc, feats, fsum, w1, b1, w2, b2, wh, bh)


def kernel(x, pos, W_qkv, W_pos0, b_pos0, W_pos1, b_pos1,
           msf_proj_w, msf_proj_b, msf_fc1_w, msf_fc1_b,
           msf_fc2_w, msf_fc2_b, msf_head_w, msf_head_b):
    B, N, DIM = x.shape
    xf = x.reshape(B * N, DIM)
    pos_t = pos.transpose(0, 2, 1)
    pospad = jnp.pad(pos.reshape(B * N, 3), ((0, 0), (0, 125)))

    q, t16, t32 = _qkv_call(xf, W_qkv, pospad)
    wp0 = jnp.pad(W_pos0, ((0, 125), (0, 0)))
    wp1 = jnp.pad(W_pos1, ((0, 125), (0, 0)))

    outs = []
    for b in range(B):
        idx32_mat = _topk_call(pos, pos_t, b)
        idx16 = idx32_mat[:, :, :16].reshape(-1)
        idx32 = idx32_mat.reshape(-1)
        g0, g1 = _sc_gather_call(t16, t32, idx16, idx32)
        xc, feats, fsum = _attn_call(
            q, g0, g1, pospad,
            wp0, b_pos0.reshape(1, -1), wp1, b_pos1.reshape(1, -1),
            msf_proj_w, msf_proj_b.reshape(1, -1), N, b)
        outs.append(_msf_call(xc, feats, fsum,
                              msf_fc1_w, msf_fc1_b.reshape(1, -1),
                              msf_fc2_w, msf_fc2_b.reshape(1, -1),
                              msf_head_w, msf_head_b.reshape(1, -1), N))
    return jnp.concatenate(outs, axis=0).reshape(B, N, DIM)


# speedup vs baseline: 4.4997x; 1.0059x over previous
import functools

import jax
import jax.numpy as jnp
from jax import lax
from jax.experimental import pallas as pl
from jax.experimental.pallas import tpu as pltpu
from jax.experimental.pallas import tpu_sc as plsc

_NUM_NEI = (16, 32)
_HD = 64
_SCALE = _HD ** (-0.5)

_NW = 32
_CH = 64


def _gelu(x):
    return 0.5 * x * (1.0 + lax.erf(x * (2.0 ** -0.5)))


def _mm(a, b):
    return lax.dot_general(a, b, (((1,), (0,)), ((), ())),
                           preferred_element_type=jnp.float32)


def _topk_body(pos_ref, pos_t_ref, idx_ref, *, blkq, n, k, b):
    qp = pos_ref[0]
    pt = pos_t_ref[0]
    kn = (pt * pt).sum(axis=0, keepdims=True)
    scores = jnp.broadcast_to(kn, (blkq, n))
    for c in range(3):
        scores = scores - 2.0 * (qp[:, c:c + 1] * pt[c:c + 1, :])
    iota = lax.broadcasted_iota(jnp.int32, (blkq, n), 1).astype(jnp.float32)
    cols = []
    big = jnp.float32(jnp.inf)
    fn = jnp.float32(n)
    for _ in range(k):
        m = jnp.min(scores, axis=1, keepdims=True)
        sel = jnp.where(scores == m, iota, fn)
        idx = jnp.min(sel, axis=1, keepdims=True)
        cols.append(idx)
        scores = jnp.where(iota == idx, big, scores)
    mat = jnp.concatenate(cols, axis=1).astype(jnp.int32)
    idx_ref[0] = mat + b * n


def _topk_call(pos, pos_t, b):
    B, N, _ = pos.shape
    blkq = 256
    nb = N // blkq
    k = _NUM_NEI[1]
    return pl.pallas_call(
        functools.partial(_topk_body, blkq=blkq, n=N, k=k, b=b),
        grid=(nb,),
        in_specs=[
            pl.BlockSpec((1, blkq, 3), lambda i, b=b: (b, i, 0)),
            pl.BlockSpec((1, 3, N), lambda i, b=b: (b, 0, 0)),
        ],
        out_specs=pl.BlockSpec((1, blkq, k), lambda i: (0, i, 0)),
        out_shape=jax.ShapeDtypeStruct((1, N, k), jnp.int32),
    )(pos, pos_t)


def _pack_kv(kf, vf):
    kb = lax.bitcast_convert_type(kf.astype(jnp.bfloat16).astype(jnp.float32),
                                  jnp.int32)
    vb = lax.bitcast_convert_type(vf.astype(jnp.bfloat16).astype(jnp.float32),
                                  jnp.int32)
    return jnp.bitwise_or(lax.shift_right_logical(kb, 16), vb)


def _qkv_body(x_ref, w_ref, pp_ref, q_ref, t16_ref, t32_ref):
    y = _mm(x_ref[...], w_ref[...])
    q_ref[...] = y[:, 0:256]
    t16_ref[...] = _pack_kv(y[:, 256:384], y[:, 512:640])
    t32_ref[:, 0:128] = _pack_kv(y[:, 384:512], y[:, 640:768])
    t32_ref[:, 128:256] = lax.bitcast_convert_type(pp_ref[...], jnp.int32)


def _qkv_call(xf, w_qkv, pospad):
    M, D = xf.shape
    blk = 512
    nb = M // blk
    outs = [
        jax.ShapeDtypeStruct((M, 256), jnp.float32),
        jax.ShapeDtypeStruct((M, 128), jnp.int32),
        jax.ShapeDtypeStruct((M, 256), jnp.int32),
    ]
    return pl.pallas_call(
        _qkv_body,
        grid=(nb,),
        in_specs=[
            pl.BlockSpec((blk, D), lambda i: (i, 0)),
            pl.BlockSpec((D, 768), lambda i: (0, 0)),
            pl.BlockSpec((blk, 128), lambda i: (i, 0)),
        ],
        out_specs=[
            pl.BlockSpec((blk, 256), lambda i: (i, 0)),
            pl.BlockSpec((blk, 128), lambda i: (i, 0)),
            pl.BlockSpec((blk, 256), lambda i: (i, 0)),
        ],
        out_shape=outs,
    )(xf, w_qkv, pospad)


def _pipe_gather(tbl, idxv, out_ref, bufa, bufb, sema, semb, nchunk, row_base):
    def fire(c, buf, sem):
        pltpu.async_copy(tbl.at[idxv.at[c]], buf, sem)

    def drain(buf, sem):
        pltpu.make_async_copy(tbl.at[idxv.at[0]], buf, sem).wait()

    def scat(c, buf):
        pltpu.sync_copy(buf, out_ref.at[pl.ds(row_base + c * _CH, _CH)])

    fire(0, bufa, sema)

    def body(p, carry):
        c0 = 2 * p
        fire(c0 + 1, bufb, semb)
        drain(bufa, sema)
        scat(c0, bufa)
        fire(c0 + 2, bufa, sema)
        drain(bufb, semb)
        scat(c0 + 1, bufb)
        return carry

    lax.fori_loop(0, nchunk // 2 - 1, body, 0)
    c0 = nchunk - 2
    fire(c0 + 1, bufb, semb)
    drain(bufa, sema)
    scat(c0, bufa)
    drain(bufb, semb)
    scat(c0 + 1, bufb)


def _sc_gather_body(t16, t32, idx16r, idx32r, g0out, g1out,
                    idxv16, idxv32, bufa16, bufb16, bufa32, bufb32,
                    sema, semb, semc, semd, *, c16, c32):
    wid = lax.axis_index("s") * 2 + lax.axis_index("c")
    pltpu.sync_copy(idx16r.at[pl.ds(wid * c16, c16)], idxv16)
    pltpu.sync_copy(idx32r.at[pl.ds(wid * c32, c32)], idxv32)
    _pipe_gather(t16, idxv16, g0out, bufa16, bufb16, sema, semb,
                 c16, wid * c16 * _CH)
    _pipe_gather(t32, idxv32, g1out, bufa32, bufb32, semc, semd,
                 c32, wid * c32 * _CH)


def _sc_gather_call(t16, t32, idx16, idx32):
    n16 = idx16.shape[0]
    n32 = idx32.shape[0]
    c16 = n16 // (_NW * _CH)
    c32 = n32 // (_NW * _CH)
    idx16r = idx16.reshape(-1, _CH)
    idx32r = idx32.reshape(-1, _CH)
    mesh = plsc.VectorSubcoreMesh(core_axis_name="c", subcore_axis_name="s",
                                  num_cores=2, num_subcores=16)
    fn = pl.kernel(
        functools.partial(_sc_gather_body, c16=c16, c32=c32),
        out_type=(
            jax.ShapeDtypeStruct((n16, 128), jnp.int32),
            jax.ShapeDtypeStruct((n32, 256), jnp.int32),
        ),
        mesh=mesh,
        scratch_types=[
            pltpu.VMEM((c16, _CH), jnp.int32),
            pltpu.VMEM((c32, _CH), jnp.int32),
            pltpu.VMEM((_CH, 128), jnp.int32),
            pltpu.VMEM((_CH, 128), jnp.int32),
            pltpu.VMEM((_CH, 256), jnp.int32),
            pltpu.VMEM((_CH, 256), jnp.int32),
            pltpu.SemaphoreType.DMA,
            pltpu.SemaphoreType.DMA,
            pltpu.SemaphoreType.DMA,
            pltpu.SemaphoreType.DMA,
        ],
    )
    return fn(t16, t32, idx16r, idx32r)


def _attn_body(q_ref, g0_ref, g1_ref, pos_ref,
               wp0_ref, bp0_ref, wp1_ref, bp1_ref, wproj_ref, bproj_ref,
               xc_ref, feats_ref, fsum_ref, *, blk):
    i = pl.program_id(0)
    q = q_ref[...]
    pq = pos_ref[...]
    g0i = g0_ref[...]
    g1i = g1_ref[...]
    kk0 = lax.bitcast_convert_type(lax.shift_left(g0i, 16), jnp.float32)
    vv0 = lax.bitcast_convert_type(
        jnp.bitwise_and(g0i, jnp.int32(-65536)), jnp.float32)
    g1kv = g1i[:, 0:128]
    kk1 = lax.bitcast_convert_type(lax.shift_left(g1kv, 16), jnp.float32)
    vv1 = lax.bitcast_convert_type(
        jnp.bitwise_and(g1kv, jnp.int32(-65536)), jnp.float32)
    xyzr = lax.bitcast_convert_type(g1i[:, 128:256],
                                    jnp.float32).reshape(blk, 32, 128)

    lane = lax.broadcasted_iota(jnp.int32, (128, 8), 0)
    colj = lax.broadcasted_iota(jnp.int32, (128, 8), 1)
    seg = jnp.where((lane < 64) == (colj == 0), 1.0, 0.0) * jnp.where(colj < 2, 1.0, 0.0)
    rowi = lax.broadcasted_iota(jnp.int32, (8, 128), 0)
    lanej = lax.broadcasted_iota(jnp.int32, (8, 128), 1)
    segt = jnp.where((lanej < 64) == (rowi == 0), 1.0, 0.0) * jnp.where(rowi < 2, 1.0, 0.0)

    relf = (pq[:, None, :] - xyzr).reshape(blk * 32, 128)

    vrp1 = _gelu(_mm(relf, wp1_ref[...]) + bp1_ref[...])
    rel0 = relf.reshape(blk, 32, 128)[:, :16, :].reshape(blk * 16, 128)
    vrp0 = _gelu(_mm(rel0, wp0_ref[...]) + bp0_ref[...])

    outs = []
    for g, (kk2, vv2, vrp, nk) in enumerate((
            (kk0, vv0, vrp0, 16),
            (kk1, vv1, vrp1, 32))):
        qg = q[:, g * 128:(g + 1) * 128]
        prod = (qg[:, None, :] * kk2.reshape(blk, nk, 128)).reshape(blk * nk, 128)
        logits8 = _mm(prod * _SCALE + vrp, seg).reshape(blk, nk, 8)
        m = jnp.max(logits8, axis=1, keepdims=True)
        e = jnp.exp(logits8 - m)
        w3 = e * (1.0 / e.sum(axis=1, keepdims=True))
        wb = _mm(w3.reshape(blk * nk, 8), segt)
        contrib = (wb * (vv2 + vrp)).reshape(blk, nk, 128)
        outs.append(contrib.sum(axis=1))

    xc = jnp.concatenate(outs, axis=1)
    feats = _gelu(_mm(xc, wproj_ref[...]) + bproj_ref[...])
    xc_ref[...] = xc
    feats_ref[...] = feats
    part = jnp.broadcast_to(feats.sum(axis=0, keepdims=True)[None], (1, 8, 256))

    @pl.when(i == 0)
    def _():
        fsum_ref[...] = part

    @pl.when(i != 0)
    def _():
        fsum_ref[...] = fsum_ref[...] + part


def _attn_call(q, g0, g1, pospad, w_pos0, b_pos0, w_pos1, b_pos1,
               wproj, bproj, N, b):
    blk = 128
    nb = N // blk
    outs = [
        jax.ShapeDtypeStruct((N, 256), jnp.float32),
        jax.ShapeDtypeStruct((N, 256), jnp.float32),
        jax.ShapeDtypeStruct((1, 8, 256), jnp.float32),
    ]
    return pl.pallas_call(
        functools.partial(_attn_body, blk=blk),
        grid=(nb,),
        in_specs=[
            pl.BlockSpec((blk, 256), lambda i, b=b: (b * (N // 128) + i, 0)),
            pl.BlockSpec((blk * 16, 128), lambda i: (i, 0)),
            pl.BlockSpec((blk * 32, 256), lambda i: (i, 0)),
            pl.BlockSpec((blk, 128), lambda i, b=b: (b * (N // 128) + i, 0)),
            pl.BlockSpec((128, 128), lambda i: (0, 0)),
            pl.BlockSpec((1, 128), lambda i: (0, 0)),
            pl.BlockSpec((128, 128), lambda i: (0, 0)),
            pl.BlockSpec((1, 128), lambda i: (0, 0)),
            pl.BlockSpec((256, 256), lambda i: (0, 0)),
            pl.BlockSpec((1, 256), lambda i: (0, 0)),
        ],
        out_specs=[
            pl.BlockSpec((blk, 256), lambda i: (i, 0)),
            pl.BlockSpec((blk, 256), lambda i: (i, 0)),
            pl.BlockSpec((1, 8, 256), lambda i: (0, 0, 0)),
        ],
        out_shape=outs,
    )(q, g0, g1, pospad, w_pos0, b_pos0, w_pos1, b_pos1, wproj, bproj)


def _msf_body(xc_ref, feats_ref, fsum_ref, w1_ref, b1_ref, w2_ref, b2_ref,
              wh_ref, bh_ref, out_ref, *, n):
    s = fsum_ref[0, 0:1, :] * (1.0 / n)
    z = _gelu(_mm(s, w1_ref[...]) + b1_ref[...])
    al = _mm(z, w2_ref[...]) + b2_ref[...]
    a0 = al[:, :128]
    a1 = al[:, 128:]
    m = jnp.maximum(a0, a1)
    e0 = jnp.exp(a0 - m)
    e1 = jnp.exp(a1 - m)
    tot = e0 + e1
    av = jnp.concatenate([e0 / tot, e1 / tot], axis=1)
    scaled = xc_ref[...] * av
    out_ref[...] = feats_ref[...] + _mm(scaled, wh_ref[...]) + bh_ref[...]


def _msf_call(xc, feats, fsum, w1, b1, w2, b2, wh, bh, N):
    blk = 512
    nb = N // blk
    return pl.pallas_call(
        functools.partial(_msf_body, n=N),
        grid=(nb,),
        in_specs=[
            pl.BlockSpec((blk, 256), lambda i: (i, 0)),
            pl.BlockSpec((blk, 256), lambda i: (i, 0)),
            pl.BlockSpec((1, 8, 256), lambda i: (0, 0, 0)),
            pl.BlockSpec((256, 128), lambda i: (0, 0)),
            pl.BlockSpec((1, 128), lambda i: (0, 0)),
            pl.BlockSpec((128, 256), lambda i: (0, 0)),
            pl.BlockSpec((1, 256), lambda i: (0, 0)),
            pl.BlockSpec((256, 256), lambda i: (0, 0)),
            pl.BlockSpec((1, 256), lambda i: (0, 0)),
        ],
        out_specs=pl.BlockSpec((blk, 256), lambda i: (i, 0)),
        out_shape=jax.ShapeDtypeStruct((N, 256), jnp.float32),
    )(xc, feats, fsum, w1, b1, w2, b2, wh, bh)


def kernel(x, pos, W_qkv, W_pos0, b_pos0, W_pos1, b_pos1,
           msf_proj_w, msf_proj_b, msf_fc1_w, msf_fc1_b,
           msf_fc2_w, msf_fc2_b, msf_head_w, msf_head_b):
    B, N, DIM = x.shape
    xf = x.reshape(B * N, DIM)
    pos_t = pos.transpose(0, 2, 1)
    pospad = jnp.pad(pos.reshape(B * N, 3), ((0, 0), (0, 125)))

    q, t16, t32 = _qkv_call(xf, W_qkv, pospad)
    wp0 = jnp.pad(W_pos0, ((0, 125), (0, 0)))
    wp1 = jnp.pad(W_pos1, ((0, 125), (0, 0)))

    outs = []
    for b in range(B):
        idx32_mat = _topk_call(pos, pos_t, b)
        idx16 = idx32_mat[:, :, :16].reshape(-1)
        idx32 = idx32_mat.reshape(-1)
        g0, g1 = _sc_gather_call(t16, t32, idx16, idx32)
        xc, feats, fsum = _attn_call(
            q, g0, g1, pospad,
            wp0, b_pos0.reshape(1, -1), wp1, b_pos1.reshape(1, -1),
            msf_proj_w, msf_proj_b.reshape(1, -1), N, b)
        outs.append(_msf_call(xc, feats, fsum,
                              msf_fc1_w, msf_fc1_b.reshape(1, -1),
                              msf_fc2_w, msf_fc2_b.reshape(1, -1),
                              msf_head_w, msf_head_b.reshape(1, -1), N))
    return jnp.concatenate(outs, axis=0).reshape(B, N, DIM)


# speedup vs baseline: 4.5277x; 1.0062x over previous
import functools

import jax
import jax.numpy as jnp
from jax import lax
from jax.experimental import pallas as pl
from jax.experimental.pallas import tpu as pltpu
from jax.experimental.pallas import tpu_sc as plsc

_NUM_NEI = (16, 32)
_HD = 64
_SCALE = _HD ** (-0.5)

_NW = 32
_CH = 64


def _gelu(x):
    return 0.5 * x * (1.0 + lax.erf(x * (2.0 ** -0.5)))


def _mm(a, b):
    return lax.dot_general(a, b, (((1,), (0,)), ((), ())),
                           preferred_element_type=jnp.float32)


def _topk_body(pos_ref, pos_t_ref, idx_ref, *, blkq, n, k, b):
    qp = pos_ref[0]
    pt = pos_t_ref[0]
    kn = (pt * pt).sum(axis=0, keepdims=True)
    scores = jnp.broadcast_to(kn, (blkq, n))
    for c in range(3):
        scores = scores - 2.0 * (qp[:, c:c + 1] * pt[c:c + 1, :])
    iota = lax.broadcasted_iota(jnp.int32, (blkq, n), 1).astype(jnp.float32)
    cols = []
    big = jnp.float32(jnp.inf)
    fn = jnp.float32(n)
    for _ in range(k):
        m = jnp.min(scores, axis=1, keepdims=True)
        sel = jnp.where(scores == m, iota, fn)
        idx = jnp.min(sel, axis=1, keepdims=True)
        cols.append(idx)
        scores = jnp.where(iota == idx, big, scores)
    mat = jnp.concatenate(cols, axis=1).astype(jnp.int32)
    idx_ref[0] = mat + b * n


def _topk_call(pos, pos_t, b):
    B, N, _ = pos.shape
    blkq = 256
    nb = N // blkq
    k = _NUM_NEI[1]
    return pl.pallas_call(
        functools.partial(_topk_body, blkq=blkq, n=N, k=k, b=b),
        grid=(nb,),
        in_specs=[
            pl.BlockSpec((1, blkq, 3), lambda i, b=b: (b, i, 0)),
            pl.BlockSpec((1, 3, N), lambda i, b=b: (b, 0, 0)),
        ],
        out_specs=pl.BlockSpec((1, blkq, k), lambda i: (0, i, 0)),
        out_shape=jax.ShapeDtypeStruct((1, N, k), jnp.int32),
    )(pos, pos_t)


def _pack_kv(kf, vf):
    kb = lax.bitcast_convert_type(kf.astype(jnp.bfloat16).astype(jnp.float32),
                                  jnp.int32)
    vb = lax.bitcast_convert_type(vf.astype(jnp.bfloat16).astype(jnp.float32),
                                  jnp.int32)
    return jnp.bitwise_or(lax.shift_right_logical(kb, 16), vb)


def _qkv_body(x_ref, w_ref, pp_ref, q_ref, t16_ref, t32_ref):
    y = _mm(x_ref[...], w_ref[...])
    q_ref[...] = y[:, 0:256]
    t16_ref[...] = _pack_kv(y[:, 256:384], y[:, 512:640])
    t32_ref[:, 0:128] = _pack_kv(y[:, 384:512], y[:, 640:768])
    t32_ref[:, 128:256] = lax.bitcast_convert_type(pp_ref[...], jnp.int32)


def _qkv_call(xf, w_qkv, pospad):
    M, D = xf.shape
    blk = 512
    nb = M // blk
    outs = [
        jax.ShapeDtypeStruct((M, 256), jnp.float32),
        jax.ShapeDtypeStruct((M, 128), jnp.int32),
        jax.ShapeDtypeStruct((M, 256), jnp.int32),
    ]
    return pl.pallas_call(
        _qkv_body,
        grid=(nb,),
        in_specs=[
            pl.BlockSpec((blk, D), lambda i: (i, 0)),
            pl.BlockSpec((D, 768), lambda i: (0, 0)),
            pl.BlockSpec((blk, 128), lambda i: (i, 0)),
        ],
        out_specs=[
            pl.BlockSpec((blk, 256), lambda i: (i, 0)),
            pl.BlockSpec((blk, 128), lambda i: (i, 0)),
            pl.BlockSpec((blk, 256), lambda i: (i, 0)),
        ],
        out_shape=outs,
    )(xf, w_qkv, pospad)


def _pipe_gather(tbl, idxv, out_ref, bufa, bufb, sema, semb, nchunk, row_base):
    def fire(c, buf, sem):
        pltpu.async_copy(tbl.at[idxv.at[c]], buf, sem)

    def drain(buf, sem):
        pltpu.make_async_copy(tbl.at[idxv.at[0]], buf, sem).wait()

    def scat(c, buf):
        pltpu.sync_copy(buf, out_ref.at[pl.ds(row_base + c * _CH, _CH)])

    fire(0, bufa, sema)

    def body(p, carry):
        c0 = 2 * p
        fire(c0 + 1, bufb, semb)
        drain(bufa, sema)
        scat(c0, bufa)
        fire(c0 + 2, bufa, sema)
        drain(bufb, semb)
        scat(c0 + 1, bufb)
        return carry

    lax.fori_loop(0, nchunk // 2 - 1, body, 0)
    c0 = nchunk - 2
    fire(c0 + 1, bufb, semb)
    drain(bufa, sema)
    scat(c0, bufa)
    drain(bufb, semb)
    scat(c0 + 1, bufb)


def _sc_gather_body(t16, t32, idx16r, idx32r, g0out, g1out,
                    idxv16, idxv32, bufa16, bufb16, bufa32, bufb32,
                    sema, semb, semc, semd, *, c16, c32):
    wid = lax.axis_index("s") * 2 + lax.axis_index("c")
    pltpu.sync_copy(idx16r.at[pl.ds(wid * c16, c16)], idxv16)
    pltpu.sync_copy(idx32r.at[pl.ds(wid * c32, c32)], idxv32)
    _pipe_gather(t16, idxv16, g0out, bufa16, bufb16, sema, semb,
                 c16, wid * c16 * _CH)
    _pipe_gather(t32, idxv32, g1out, bufa32, bufb32, semc, semd,
                 c32, wid * c32 * _CH)


def _sc_gather_call(t16, t32, idx16, idx32):
    n16 = idx16.shape[0]
    n32 = idx32.shape[0]
    c16 = n16 // (_NW * _CH)
    c32 = n32 // (_NW * _CH)
    idx16r = idx16.reshape(-1, _CH)
    idx32r = idx32.reshape(-1, _CH)
    mesh = plsc.VectorSubcoreMesh(core_axis_name="c", subcore_axis_name="s",
                                  num_cores=2, num_subcores=16)
    fn = pl.kernel(
        functools.partial(_sc_gather_body, c16=c16, c32=c32),
        out_type=(
            jax.ShapeDtypeStruct((n16, 128), jnp.int32),
            jax.ShapeDtypeStruct((n32, 256), jnp.int32),
        ),
        mesh=mesh,
        scratch_types=[
            pltpu.VMEM((c16, _CH), jnp.int32),
            pltpu.VMEM((c32, _CH), jnp.int32),
            pltpu.VMEM((_CH, 128), jnp.int32),
            pltpu.VMEM((_CH, 128), jnp.int32),
            pltpu.VMEM((_CH, 256), jnp.int32),
            pltpu.VMEM((_CH, 256), jnp.int32),
            pltpu.SemaphoreType.DMA,
            pltpu.SemaphoreType.DMA,
            pltpu.SemaphoreType.DMA,
            pltpu.SemaphoreType.DMA,
        ],
    )
    return fn(t16, t32, idx16r, idx32r)


def _attn_body(q_ref, g0_ref, g1_ref, pos_ref,
               wp0_ref, bp0_ref, wp1_ref, bp1_ref, wproj_ref, bproj_ref,
               xc_ref, feats_ref, fsum_ref, *, blk):
    i = pl.program_id(0)
    q = q_ref[...]
    pq = pos_ref[...]
    g0i = g0_ref[...]
    g1i = g1_ref[...]
    kk0 = lax.bitcast_convert_type(lax.shift_left(g0i, 16), jnp.float32)
    vv0 = lax.bitcast_convert_type(
        jnp.bitwise_and(g0i, jnp.int32(-65536)), jnp.float32)
    g1kv = g1i[:, 0:128]
    kk1 = lax.bitcast_convert_type(lax.shift_left(g1kv, 16), jnp.float32)
    vv1 = lax.bitcast_convert_type(
        jnp.bitwise_and(g1kv, jnp.int32(-65536)), jnp.float32)
    xyzr = lax.bitcast_convert_type(g1i[:, 128:256],
                                    jnp.float32).reshape(blk, 32, 128)

    li = lax.broadcasted_iota(jnp.int32, (128, 128), 0)
    lj = lax.broadcasted_iota(jnp.int32, (128, 128), 1)
    segd = jnp.where((li >= 64) == (lj >= 64), 1.0, 0.0)

    relf = (pq[:, None, :] - xyzr).reshape(blk * 32, 128)

    vrp1 = _gelu(_mm(relf, wp1_ref[...]) + bp1_ref[...])
    rel0 = relf.reshape(blk, 32, 128)[:, :16, :].reshape(blk * 16, 128)
    vrp0 = _gelu(_mm(rel0, wp0_ref[...]) + bp0_ref[...])

    outs = []
    for g, (kk2, vv2, vrp, nk) in enumerate((
            (kk0, vv0, vrp0, 16),
            (kk1, vv1, vrp1, 32))):
        qg = q[:, g * 128:(g + 1) * 128]
        prod = (qg[:, None, :] * kk2.reshape(blk, nk, 128)).reshape(blk * nk, 128)
        l3 = _mm(prod * _SCALE + vrp, segd).reshape(blk, nk, 128)
        m = jnp.max(l3, axis=1, keepdims=True)
        e = jnp.exp(l3 - m)
        w = e * (1.0 / e.sum(axis=1, keepdims=True))
        contrib = w * (vv2 + vrp).reshape(blk, nk, 128)
        outs.append(contrib.sum(axis=1))

    xc = jnp.concatenate(outs, axis=1)
    feats = _gelu(_mm(xc, wproj_ref[...]) + bproj_ref[...])
    xc_ref[...] = xc
    feats_ref[...] = feats
    part = jnp.broadcast_to(feats.sum(axis=0, keepdims=True)[None], (1, 8, 256))

    @pl.when(i == 0)
    def _():
        fsum_ref[...] = part

    @pl.when(i != 0)
    def _():
        fsum_ref[...] = fsum_ref[...] + part


def _attn_call(q, g0, g1, pospad, w_pos0, b_pos0, w_pos1, b_pos1,
               wproj, bproj, N, b):
    blk = 128
    nb = N // blk
    outs = [
        jax.ShapeDtypeStruct((N, 256), jnp.float32),
        jax.ShapeDtypeStruct((N, 256), jnp.float32),
        jax.ShapeDtypeStruct((1, 8, 256), jnp.float32),
    ]
    return pl.pallas_call(
        functools.partial(_attn_body, blk=blk),
        grid=(nb,),
        in_specs=[
            pl.BlockSpec((blk, 256), lambda i, b=b: (b * (N // 128) + i, 0)),
            pl.BlockSpec((blk * 16, 128), lambda i: (i, 0)),
            pl.BlockSpec((blk * 32, 256), lambda i: (i, 0)),
            pl.BlockSpec((blk, 128), lambda i, b=b: (b * (N // 128) + i, 0)),
            pl.BlockSpec((128, 128), lambda i: (0, 0)),
            pl.BlockSpec((1, 128), lambda i: (0, 0)),
            pl.BlockSpec((128, 128), lambda i: (0, 0)),
            pl.BlockSpec((1, 128), lambda i: (0, 0)),
            pl.BlockSpec((256, 256), lambda i: (0, 0)),
            pl.BlockSpec((1, 256), lambda i: (0, 0)),
        ],
        out_specs=[
            pl.BlockSpec((blk, 256), lambda i: (i, 0)),
            pl.BlockSpec((blk, 256), lambda i: (i, 0)),
            pl.BlockSpec((1, 8, 256), lambda i: (0, 0, 0)),
        ],
        out_shape=outs,
    )(q, g0, g1, pospad, w_pos0, b_pos0, w_pos1, b_pos1, wproj, bproj)


def _msf_body(xc_ref, feats_ref, fsum_ref, w1_ref, b1_ref, w2_ref, b2_ref,
              wh_ref, bh_ref, out_ref, *, n):
    s = fsum_ref[0, 0:1, :] * (1.0 / n)
    z = _gelu(_mm(s, w1_ref[...]) + b1_ref[...])
    al = _mm(z, w2_ref[...]) + b2_ref[...]
    a0 = al[:, :128]
    a1 = al[:, 128:]
    m = jnp.maximum(a0, a1)
    e0 = jnp.exp(a0 - m)
    e1 = jnp.exp(a1 - m)
    tot = e0 + e1
    av = jnp.concatenate([e0 / tot, e1 / tot], axis=1)
    scaled = xc_ref[...] * av
    out_ref[...] = feats_ref[...] + _mm(scaled, wh_ref[...]) + bh_ref[...]


def _msf_call(xc, feats, fsum, w1, b1, w2, b2, wh, bh, N):
    blk = 512
    nb = N // blk
    return pl.pallas_call(
        functools.partial(_msf_body, n=N),
        grid=(nb,),
        in_specs=[
            pl.BlockSpec((blk, 256), lambda i: (i, 0)),
            pl.BlockSpec((blk, 256), lambda i: (i, 0)),
            pl.BlockSpec((1, 8, 256), lambda i: (0, 0, 0)),
            pl.BlockSpec((256, 128), lambda i: (0, 0)),
            pl.BlockSpec((1, 128), lambda i: (0, 0)),
            pl.BlockSpec((128, 256), lambda i: (0, 0)),
            pl.BlockSpec((1, 256), lambda i: (0, 0)),
            pl.BlockSpec((256, 256), lambda i: (0, 0)),
            pl.BlockSpec((1, 256), lambda i: (0, 0)),
        ],
        out_specs=pl.BlockSpec((blk, 256), lambda i: (i, 0)),
        out_shape=jax.ShapeDtypeStruct((N, 256), jnp.float32),
    )(xc, feats, fsum, w1, b1, w2, b2, wh, bh)


def kernel(x, pos, W_qkv, W_pos0, b_pos0, W_pos1, b_pos1,
           msf_proj_w, msf_proj_b, msf_fc1_w, msf_fc1_b,
           msf_fc2_w, msf_fc2_b, msf_head_w, msf_head_b):
    B, N, DIM = x.shape
    xf = x.reshape(B * N, DIM)
    pos_t = pos.transpose(0, 2, 1)
    pospad = jnp.pad(pos.reshape(B * N, 3), ((0, 0), (0, 125)))

    q, t16, t32 = _qkv_call(xf, W_qkv, pospad)
    wp0 = jnp.pad(W_pos0, ((0, 125), (0, 0)))
    wp1 = jnp.pad(W_pos1, ((0, 125), (0, 0)))

    outs = []
    for b in range(B):
        idx32_mat = _topk_call(pos, pos_t, b)
        idx16 = idx32_mat[:, :, :16].reshape(-1)
        idx32 = idx32_mat.reshape(-1)
        g0, g1 = _sc_gather_call(t16, t32, idx16, idx32)
        xc, feats, fsum = _attn_call(
            q, g0, g1, pospad,
            wp0, b_pos0.reshape(1, -1), wp1, b_pos1.reshape(1, -1),
            msf_proj_w, msf_proj_b.reshape(1, -1), N, b)
        outs.append(_msf_call(xc, feats, fsum,
                              msf_fc1_w, msf_fc1_b.reshape(1, -1),
                              msf_fc2_w, msf_fc2_b.reshape(1, -1),
                              msf_head_w, msf_head_b.reshape(1, -1), N))
    return jnp.concatenate(outs, axis=0).reshape(B, N, DIM)
